# baseline (speedup 1.0000x reference)
import functools

import jax
import jax.numpy as jnp
from jax import lax
from jax.experimental import pallas as pl
from jax.experimental.pallas import tpu as pltpu
from jax.experimental.pallas import tpu_sc as plsc

B = 16384
EMB = 64
L = 16

_info = plsc.get_sparse_core_info()
_NC, _NS = _info.num_cores, _info.num_subcores
NW = _NC * _NS
BPW = B // NW
CH = 128
NCH = BPW // CH

_mesh = plsc.VectorSubcoreMesh(core_axis_name="c", subcore_axis_name="s")


@functools.partial(
    pl.kernel,
    mesh=_mesh,
    compiler_params=pltpu.CompilerParams(
        needs_layout_passes=False, use_tc_tiling_on_sc=False),
    out_type=jax.ShapeDtypeStruct((B,), jnp.float32),
    scratch_types=[
        pltpu.VMEM((NCH, CH), jnp.int32),
        pltpu.VMEM((NCH, CH), jnp.int32),
        pltpu.VMEM((NCH, CH), jnp.int32),
        pltpu.VMEM((BPW, EMB), jnp.float32),
        pltpu.VMEM((BPW, EMB), jnp.float32),
        pltpu.VMEM((BPW, EMB), jnp.float32),
        pltpu.VMEM((BPW,), jnp.float32),
        pltpu.SemaphoreType.DMA,
        pltpu.SemaphoreType.DMA,
        pltpu.SemaphoreType.DMA,
    ],
)
def _lp_kernel(s_hbm, p_hbm, o_hbm, node_hbm, rel_hbm, out_hbm,
               sidx_v, pidx_v, oidx_v, srow_v, prow_v, orow_v, out_v,
               sem_s, sem_p, sem_o):
    wid = lax.axis_index("s") * _NC + lax.axis_index("c")
    base = wid * NCH

    pltpu.sync_copy(s_hbm.at[pl.ds(base, NCH)], sidx_v)
    pltpu.sync_copy(p_hbm.at[pl.ds(base, NCH)], pidx_v)
    pltpu.sync_copy(o_hbm.at[pl.ds(base, NCH)], oidx_v)

    copies = []
    for j in range(NCH):
        rows = pl.ds(j * CH, CH)
        copies.append(pltpu.async_copy(
            node_hbm.at[sidx_v.at[j]], srow_v.at[rows], sem_s))
        copies.append(pltpu.async_copy(
            rel_hbm.at[pidx_v.at[j]], prow_v.at[rows], sem_p))
        copies.append(pltpu.async_copy(
            node_hbm.at[oidx_v.at[j]], orow_v.at[rows], sem_o))
    for c in copies:
        c.wait()

    lane = lax.iota(jnp.int32, 16)

    def batch_body(b, carry):
        rows = b * L + lane
        acc = jnp.zeros((L,), jnp.float32)
        for d in range(EMB):
            col = jnp.full((L,), d, jnp.int32)
            sv = plsc.load_gather(srow_v, [rows, col])
            pv = plsc.load_gather(prow_v, [rows, col])
            ov = plsc.load_gather(orow_v, [rows, col])
            acc = acc + sv * pv * ov
        out_v[pl.ds(b * L, L)] = 1.0 / (1.0 + jnp.exp(-acc))
        return carry

    lax.fori_loop(0, BPW // L, batch_body, 0)

    pltpu.sync_copy(out_v, out_hbm.at[pl.ds(wid * BPW, BPW)])


def kernel(s_idx, p_idx, o_idx, node_emb, rel_emb):
    s2 = s_idx.reshape(B // CH, CH)
    p2 = p_idx.reshape(B // CH, CH)
    o2 = o_idx.reshape(B // CH, CH)
    return _lp_kernel(s2, p2, o2, node_emb, rel_emb)


# speedup vs baseline: 1.0128x; 1.0128x over previous
import functools

import jax
import jax.numpy as jnp
from jax import lax
from jax.experimental import pallas as pl
from jax.experimental.pallas import tpu as pltpu
from jax.experimental.pallas import tpu_sc as plsc

B = 16384
EMB = 64
L = 16
SUB = 8
RPT = 16

_info = plsc.get_sparse_core_info()
_NC, _NS = _info.num_cores, _info.num_subcores
NW = _NC * _NS
BPW = B // NW
NCHUNK = BPW // L
NTILE = 1000000 // RPT
RTILE = 128 // RPT

_mesh = plsc.VectorSubcoreMesh(core_axis_name="c", subcore_axis_name="s")


@functools.partial(
    pl.kernel,
    mesh=_mesh,
    compiler_params=pltpu.CompilerParams(needs_layout_passes=False),
    out_type=jax.ShapeDtypeStruct((B,), jnp.float32),
    scratch_types=[
        pltpu.VMEM((BPW // 128, 128), jnp.int32),
        pltpu.VMEM((BPW // 128, 128), jnp.int32),
        pltpu.VMEM((BPW // 128, 128), jnp.int32),
        pltpu.VMEM((NCHUNK, L), jnp.int32),
        pltpu.VMEM((NCHUNK, L), jnp.int32),
        pltpu.VMEM((NCHUNK, L), jnp.int32),
        pltpu.VMEM((NCHUNK, L), jnp.int32),
        pltpu.VMEM((NCHUNK, L), jnp.int32),
        pltpu.VMEM((NCHUNK, L), jnp.int32),
        pltpu.VMEM((NCHUNK, L), jnp.int32),
        pltpu.VMEM((NCHUNK, L), jnp.int32),
        pltpu.VMEM((NCHUNK, L), jnp.int32),
        pltpu.VMEM((L, SUB, 128), jnp.float32),
        pltpu.VMEM((L, SUB, 128), jnp.float32),
        pltpu.VMEM((L, SUB, 128), jnp.float32),
        pltpu.VMEM((L, SUB, 128), jnp.float32),
        pltpu.VMEM((RTILE, SUB, 128), jnp.float32),
        pltpu.VMEM((BPW,), jnp.float32),
        pltpu.SemaphoreType.DMA,
        pltpu.SemaphoreType.DMA,
    ],
)
def _lp_kernel(s_hbm, p_hbm, o_hbm, node_hbm, rel_hbm, out_hbm,
               sidx_v, pidx_v, oidx_v,
               stile_v, ssub_v, otile_v, osub_v, ptile_v, psub_v,
               scol_v, ocol_v, pcol_v,
               sbuf0, sbuf1, obuf0, obuf1, rel_l, out_v,
               sem_s, sem_o):
    wid = lax.axis_index("s") * _NC + lax.axis_index("c")

    pltpu.sync_copy(s_hbm.at[wid], sidx_v)
    pltpu.sync_copy(p_hbm.at[wid], pidx_v)
    pltpu.sync_copy(o_hbm.at[wid], oidx_v)
    pltpu.sync_copy(rel_hbm, rel_l)

    for b in range(NCHUNK):
        r, k = divmod(b, 128 // L)
        sl = pl.ds(k * L, L)
        sv = sidx_v[r, sl]
        stile_v[b, :] = sv >> 4
        ssub_v[b, :] = (sv & 15) >> 1
        scol_v[b, :] = (sv & 1) << 6
        ov = oidx_v[r, sl]
        otile_v[b, :] = ov >> 4
        osub_v[b, :] = (ov & 15) >> 1
        ocol_v[b, :] = (ov & 1) << 6
        pv = pidx_v[r, sl]
        ptile_v[b, :] = pv >> 4
        psub_v[b, :] = (pv & 15) >> 1
        pcol_v[b, :] = (pv & 1) << 6

    def issue(c, sdst, odst):
        pltpu.async_copy(node_hbm.at[stile_v.at[c]], sdst, sem_s)
        pltpu.async_copy(node_hbm.at[otile_v.at[c]], odst, sem_o)

    def wait_pair(sdst, odst):
        pltpu.make_async_copy(node_hbm.at[stile_v.at[0]], sdst, sem_s).wait()
        pltpu.make_async_copy(node_hbm.at[otile_v.at[0]], odst, sem_o).wait()

    lane = lax.iota(jnp.int32, 16)

    def compute(c, sbufx, obufx):
        ssub = ssub_v[c, :]
        osub = osub_v[c, :]
        scol = scol_v[c, :]
        ocol = ocol_v[c, :]
        ptile = ptile_v[c, :]
        psub = psub_v[c, :]
        pcol = pcol_v[c, :]
        acc = jnp.zeros((L,), jnp.float32)
        for d in range(EMB):
            svv = plsc.load_gather(sbufx, [lane, ssub, scol + d])
            ovv = plsc.load_gather(obufx, [lane, osub, ocol + d])
            pvv = plsc.load_gather(rel_l, [ptile, psub, pcol + d])
            acc = acc + svv * pvv * ovv
        out_v[pl.ds(c * L, L)] = 1.0 / (1.0 + jnp.exp(-acc))

    issue(0, sbuf0, obuf0)

    def loop_body(g, carry):
        c0 = 2 * g
        issue(c0 + 1, sbuf1, obuf1)
        wait_pair(sbuf0, obuf0)
        compute(c0, sbuf0, obuf0)

        @pl.when(g < NCHUNK // 2 - 1)
        def _():
            issue(c0 + 2, sbuf0, obuf0)

        wait_pair(sbuf1, obuf1)
        compute(c0 + 1, sbuf1, obuf1)
        return carry

    lax.fori_loop(0, NCHUNK // 2, loop_body, 0)

    pltpu.sync_copy(out_v, out_hbm.at[pl.ds(wid * BPW, BPW)])


def kernel(s_idx, p_idx, o_idx, node_emb, rel_emb):
    s3 = s_idx.reshape(NW, BPW // 128, 128)
    p3 = p_idx.reshape(NW, BPW // 128, 128)
    o3 = o_idx.reshape(NW, BPW // 128, 128)
    node3 = node_emb.reshape(NTILE, SUB, 128)
    rel3 = jnp.pad(rel_emb, ((0, 128 - rel_emb.shape[0]), (0, 0))).reshape(
        RTILE, SUB, 128)
    return _lp_kernel(s3, p3, o3, node3, rel3)


# speedup vs baseline: 1.0365x; 1.0234x over previous
import functools

import jax
import jax.numpy as jnp
from jax import lax
from jax.experimental import pallas as pl
from jax.experimental.pallas import tpu as pltpu
from jax.experimental.pallas import tpu_sc as plsc

B = 16384
EMB = 64
L = 16
N_REL = 100

_info = plsc.get_sparse_core_info()
_NC, _NS = _info.num_cores, _info.num_subcores
NW = _NC * _NS
BPW = B // NW
NCHUNK = BPW // L
NGRP = BPW // 128
CPG = 128 // L

_mesh = plsc.VectorSubcoreMesh(core_axis_name="c", subcore_axis_name="s")


@functools.partial(
    pl.kernel,
    mesh=_mesh,
    compiler_params=pltpu.CompilerParams(needs_layout_passes=False),
    out_type=jax.ShapeDtypeStruct((B,), jnp.float32),
    scratch_types=[
        pltpu.VMEM((NGRP, 128), jnp.int32),
        pltpu.VMEM((NGRP, 128), jnp.int32),
        pltpu.VMEM((NCHUNK, L), jnp.int32),
        pltpu.VMEM((NGRP, 128), jnp.int32),
        pltpu.VMEM((NGRP, 128), jnp.int32),
        pltpu.VMEM((NCHUNK, L), jnp.int32),
        pltpu.VMEM((NCHUNK, L), jnp.int32),
        pltpu.VMEM((128, 128), jnp.float32),
        pltpu.VMEM((128, 128), jnp.float32),
        pltpu.VMEM((128, 128), jnp.float32),
        pltpu.VMEM((128, 128), jnp.float32),
        pltpu.VMEM((N_REL // 2, 128), jnp.float32),
        pltpu.VMEM((BPW,), jnp.float32),
        pltpu.SemaphoreType.DMA,
        pltpu.SemaphoreType.DMA,
        pltpu.SemaphoreType.DMA,
        pltpu.SemaphoreType.DMA,
    ],
)
def _lp_kernel(s_hbm, p_hbm, o_hbm, node_hbm, rel_hbm, out_hbm,
               sidx_v, oidx_v, pidx_v, srid_v, orid_v, scol_v, ocol_v,
               sbuf0, sbuf1, obuf0, obuf1, rel_l, out_v,
               sem_s0, sem_s1, sem_o0, sem_o1):
    wid = lax.axis_index("s") * _NC + lax.axis_index("c")

    pltpu.sync_copy(s_hbm.at[wid], sidx_v)
    pltpu.sync_copy(o_hbm.at[wid], oidx_v)
    pltpu.sync_copy(p_hbm.at[wid], pidx_v)

    for c in range(NCHUNK):
        g, k = divmod(c, CPG)
        sl = pl.ds(k * L, L)
        sv = sidx_v[g, sl]
        srid_v[g, sl] = sv >> 1
        scol_v[c, :] = (sv & 1) << 6
        ov = oidx_v[g, sl]
        orid_v[g, sl] = ov >> 1
        ocol_v[c, :] = (ov & 1) << 6

    sbufs = (sbuf0, sbuf1)
    obufs = (obuf0, obuf1)
    ssems = (sem_s0, sem_s1)
    osems = (sem_o0, sem_o1)

    def issue(g):
        hs = pltpu.async_copy(node_hbm.at[srid_v.at[g]],
                              sbufs[g % 2], ssems[g % 2])
        ho = pltpu.async_copy(node_hbm.at[orid_v.at[g]],
                              obufs[g % 2], osems[g % 2])
        return hs, ho

    pending = issue(0)
    pltpu.sync_copy(rel_hbm, rel_l)

    lane = lax.iota(jnp.int32, L)

    for g in range(NGRP):
        nxt = issue(g + 1) if g + 1 < NGRP else None
        pending[0].wait()
        pending[1].wait()
        pending = nxt
        sb, ob = sbufs[g % 2], obufs[g % 2]

        def chunk_body(lc, carry, g=g, sb=sb, ob=ob):
            c = g * CPG + lc
            rows = lc * L + lane
            scol = scol_v[c, :]
            ocol = ocol_v[c, :]
            pvec = pidx_v[c, :]
            prow = pvec >> 1
            pcol = (pvec & 1) << 6
            acc = jnp.zeros((L,), jnp.float32)
            for d in range(EMB):
                sv = plsc.load_gather(sb, [rows, scol + d])
                ov = plsc.load_gather(ob, [rows, ocol + d])
                pv = plsc.load_gather(rel_l, [prow, pcol + d])
                acc = acc + sv * pv * ov
            out_v[pl.ds(c * L, L)] = 1.0 / (1.0 + jnp.exp(-acc))
            return carry

        lax.fori_loop(0, CPG, chunk_body, 0)

    pltpu.sync_copy(out_v, out_hbm.at[pl.ds(wid * BPW, BPW)])


def kernel(s_idx, p_idx, o_idx, node_emb, rel_emb):
    s3 = s_idx.reshape(NW, NGRP, 128)
    o3 = o_idx.reshape(NW, NGRP, 128)
    p3 = p_idx.reshape(NW, NCHUNK, L)
    node2 = node_emb.reshape(N_NODES_PACKED, 128)
    rel2 = rel_emb.reshape(N_REL // 2, 128)
    return _lp_kernel(s3, p3, o3, node2, rel2)


N_NODES_PACKED = 500000


# speedup vs baseline: 2.3721x; 2.2887x over previous
import functools

import jax
import jax.numpy as jnp
from jax import lax
from jax.experimental import pallas as pl
from jax.experimental.pallas import tpu as pltpu
from jax.experimental.pallas import tpu_sc as plsc

B = 16384
EMB = 64
L = 16
N_REL = 100

_info = plsc.get_sparse_core_info()
_NC, _NS = _info.num_cores, _info.num_subcores
NW = _NC * _NS
BPW = B // NW
NCHUNK = BPW // L
G = 64
NGRP = BPW // G
CPG = G // L

_mesh = plsc.VectorSubcoreMesh(core_axis_name="c", subcore_axis_name="s")


@functools.partial(
    pl.kernel,
    mesh=_mesh,
    compiler_params=pltpu.CompilerParams(needs_layout_passes=False),
    out_type=jax.ShapeDtypeStruct((B,), jnp.float32),
    scratch_types=[
        pltpu.VMEM((NCHUNK, L), jnp.int32),
        pltpu.VMEM((NCHUNK, L), jnp.int32),
        pltpu.VMEM((NCHUNK, L), jnp.int32),
        pltpu.VMEM((G // 8, 8, EMB), jnp.float32),
        pltpu.VMEM((G // 8, 8, EMB), jnp.float32),
        pltpu.VMEM((G // 8, 8, EMB), jnp.float32),
        pltpu.VMEM((G // 8, 8, EMB), jnp.float32),
        pltpu.VMEM((N_REL // 2, 128), jnp.float32),
        pltpu.VMEM((BPW,), jnp.float32),
        pltpu.SemaphoreType.DMA,
        pltpu.SemaphoreType.DMA,
        pltpu.SemaphoreType.DMA,
        pltpu.SemaphoreType.DMA,
    ],
)
def _lp_kernel(s_hbm, p_hbm, o_hbm, node_hbm, rel_hbm, out_hbm,
               pidx_v, sidx_v, oidx_v,
               sbuf0, sbuf1, obuf0, obuf1, rel_l, out_v,
               sem_s0, sem_s1, sem_o0, sem_o1):
    wid = lax.axis_index("s") * _NC + lax.axis_index("c")

    pltpu.sync_copy(s_hbm.at[wid], sidx_v)
    pltpu.sync_copy(o_hbm.at[wid], oidx_v)
    pltpu.sync_copy(p_hbm.at[wid], pidx_v)

    sbufs = (sbuf0, sbuf1)
    obufs = (obuf0, obuf1)
    ssems = (sem_s0, sem_s1)
    osems = (sem_o0, sem_o1)

    def issue(g, par):
        sb, ob = sbufs[par], obufs[par]
        sem_s, sem_o = ssems[par], osems[par]

        def dma_body(k, carry):
            c = g * CPG + k
            sv = sidx_v[c, :]
            ov = oidx_v[c, :]
            for j in range(L):
                i = k * L + j
                rs = sv[j]
                pltpu.async_copy(
                    node_hbm.at[pl.ds(rs >> 3, 1), pl.ds(rs & 7, 1)],
                    sb.at[pl.ds(i >> 3, 1), pl.ds(i & 7, 1)], sem_s)
                ro = ov[j]
                pltpu.async_copy(
                    node_hbm.at[pl.ds(ro >> 3, 1), pl.ds(ro & 7, 1)],
                    ob.at[pl.ds(i >> 3, 1), pl.ds(i & 7, 1)], sem_o)
            return carry

        lax.fori_loop(0, CPG, dma_body, 0)

    def wait(par):
        pltpu.make_async_copy(node_hbm.at[pl.ds(0, G // 8)],
                              sbufs[par], ssems[par]).wait()
        pltpu.make_async_copy(node_hbm.at[pl.ds(0, G // 8)],
                              obufs[par], osems[par]).wait()

    lane = lax.iota(jnp.int32, L)

    def compute(g, par):
        sb, ob = sbufs[par], obufs[par]
        for k in range(CPG):
            c_static_off = k
            c = g * CPG + c_static_off
            trip = k * L + lane
            pvec = pidx_v[c, :]
            prow = pvec >> 1
            pcol = (pvec & 1) << 6
            tq = trip >> 3
            tr = trip & 7
            acc = jnp.zeros((L,), jnp.float32)
            for d in range(EMB):
                sv = plsc.load_gather(sb, [tq, tr, lane * 0 + d])
                ov = plsc.load_gather(ob, [tq, tr, lane * 0 + d])
                pv = plsc.load_gather(rel_l, [prow, pcol + d])
                acc = acc + sv * pv * ov
            out_v[pl.ds(c * L, L)] = 1.0 / (1.0 + jnp.exp(-acc))

    issue(0, 0)
    pltpu.sync_copy(rel_hbm, rel_l)

    def pair_body(h, carry):
        g0 = 2 * h
        issue(g0 + 1, 1)
        wait(0)
        compute(g0, 0)

        @pl.when(h < NGRP // 2 - 1)
        def _():
            issue(g0 + 2, 0)

        wait(1)
        compute(g0 + 1, 1)
        return carry

    lax.fori_loop(0, NGRP // 2, pair_body, 0)

    pltpu.sync_copy(out_v, out_hbm.at[pl.ds(wid * BPW, BPW)])


def kernel(s_idx, p_idx, o_idx, node_emb, rel_emb):
    s3 = s_idx.reshape(NW, NCHUNK, L)
    o3 = o_idx.reshape(NW, NCHUNK, L)
    p3 = p_idx.reshape(NW, NCHUNK, L)
    node3 = node_emb.reshape(1000000 // 8, 8, EMB)
    rel2 = rel_emb.reshape(N_REL // 2, 128)
    return _lp_kernel(s3, p3, o3, node3, rel2)
